# baseline (speedup 1.0000x reference)
import jax
import jax.numpy as jnp
from jax.experimental import pallas as pl
from jax.experimental.pallas import tpu as pltpu

_BM = 128
_CACHE_BLOCKS = 20


def _body(bc0_ref, bc1_ref, adj_ref, bw0_ref, bw1_ref, out1_ref, fsum_ref,
          out0_ref, g_ref, cache_ref):
    phase = pl.program_id(0)
    i = pl.program_id(1)
    nrel = adj_ref.shape[0]
    bm = adj_ref.shape[1]

    @pl.when(jnp.logical_and(phase == 0, i == 0))
    def _pre():
        for r in range(nrel):
            w = bc0_ref[r, 0] * bw0_ref[0] + bc0_ref[r, 1] * bw0_ref[1]
            g_ref[r] = w.astype(jnp.bfloat16)

    @pl.when(phase == 0)
    def _layer0():
        acc = jnp.zeros((bm, out0_ref.shape[1]), jnp.float32)
        for r in range(nrel):
            ab = adj_ref[r].astype(jnp.bfloat16)

            @pl.when(i < _CACHE_BLOCKS)
            def _park():
                cache_ref[r, pl.ds(i * bm, bm), :] = ab

            acc += jnp.dot(ab, g_ref[r], preferred_element_type=jnp.float32)
        out0_ref[pl.ds(i * bm, bm), :] = acc

    @pl.when(jnp.logical_and(phase == 1, i == 0))
    def _between():
        h = jnp.maximum(out0_ref[...], 0.0)
        for r in range(nrel):
            w = bc1_ref[r, 0] * bw1_ref[0] + bc1_ref[r, 1] * bw1_ref[1]
            g_ref[r] = jnp.dot(h, w,
                               preferred_element_type=jnp.float32
                               ).astype(jnp.bfloat16)
        fsum_ref[0:1, :] = jnp.sum(out0_ref[...], axis=0, keepdims=True)
        fsum_ref[1:2, :] = jnp.zeros((1, fsum_ref.shape[1]), jnp.float32)

    def _emit_layer1(read_block):
        acc = jnp.zeros(out1_ref.shape, jnp.float32)
        for r in range(nrel):
            acc += jnp.dot(read_block(r), g_ref[r],
                           preferred_element_type=jnp.float32)
        out1_ref[...] = acc
        fsum_ref[1:2, :] += jnp.sum(acc, axis=0, keepdims=True)

    @pl.when(jnp.logical_and(phase == 1, i < _CACHE_BLOCKS))
    def _layer1_cached():
        _emit_layer1(lambda r: cache_ref[r, pl.ds(i * bm, bm), :])

    @pl.when(jnp.logical_and(phase == 1, i >= _CACHE_BLOCKS))
    def _layer1_streamed():
        _emit_layer1(lambda r: adj_ref[r].astype(jnp.bfloat16))


def kernel(adj_mat_list, bw0, bc0, bw1, bc1):
    nrel, n, _ = adj_mat_list.shape
    nb, _, h0 = bw0.shape
    h1 = bw1.shape[2]
    ni = n // _BM
    grid = (2, ni)

    def adj_index(p, i):
        return (0, jnp.where(p == 0, i, jnp.maximum(i, _CACHE_BLOCKS)), 0)

    out1, fsum = pl.pallas_call(
        _body,
        grid=grid,
        in_specs=[
            pl.BlockSpec(memory_space=pltpu.SMEM),
            pl.BlockSpec(memory_space=pltpu.SMEM),
            pl.BlockSpec((nrel, _BM, n), adj_index),
            pl.BlockSpec((nb, n, h0), lambda p, i: (0, 0, 0)),
            pl.BlockSpec((nb, h0, h1), lambda p, i: (0, 0, 0)),
        ],
        out_specs=[
            pl.BlockSpec((_BM, h1), lambda p, i: (i, 0)),
            pl.BlockSpec((2, h0), lambda p, i: (0, 0)),
        ],
        out_shape=[
            jax.ShapeDtypeStruct((n, h1), jnp.float32),
            jax.ShapeDtypeStruct((2, h0), jnp.float32),
        ],
        scratch_shapes=[
            pltpu.VMEM((n, h0), jnp.float32),
            pltpu.VMEM((nrel, n, h0), jnp.bfloat16),
            pltpu.VMEM((nrel, _CACHE_BLOCKS * _BM, n), jnp.bfloat16),
        ],
        compiler_params=pltpu.CompilerParams(
            vmem_limit_bytes=63 * 1024 * 1024,
        ),
    )(bc0, bc1, adj_mat_list, bw0, bw1)

    final_rep = fsum.reshape(1, h0 + h1)
    return (out1, final_rep)


# speedup vs baseline: 1.1361x; 1.1361x over previous
import jax
import jax.numpy as jnp
from jax.experimental import pallas as pl
from jax.experimental.pallas import tpu as pltpu

_BM = 256
_PREC = jax.lax.Precision.DEFAULT


def _body(bc0_ref, bc1_ref, adj_ref, bw0_ref, bw1_ref, out1_ref, fsum_ref,
          out0_ref, g_ref):
    phase = pl.program_id(0)
    i = pl.program_id(1)
    nrel = adj_ref.shape[0]
    bm = adj_ref.shape[1]

    @pl.when(jnp.logical_and(phase == 0, i == 0))
    def _pre():
        for r in range(nrel):
            g_ref[r] = bc0_ref[r, 0] * bw0_ref[0] + bc0_ref[r, 1] * bw0_ref[1]

    @pl.when(phase == 0)
    def _layer0():
        acc = jnp.zeros((bm, out0_ref.shape[1]), jnp.float32)
        for r in range(nrel):
            acc += jnp.dot(adj_ref[r], g_ref[r], precision=_PREC,
                           preferred_element_type=jnp.float32)
        out0_ref[pl.ds(i * bm, bm), :] = acc

    @pl.when(jnp.logical_and(phase == 1, i == 0))
    def _between():
        h = jnp.maximum(out0_ref[...], 0.0)
        for r in range(nrel):
            w = bc1_ref[r, 0] * bw1_ref[0] + bc1_ref[r, 1] * bw1_ref[1]
            g_ref[r] = jnp.dot(h, w, preferred_element_type=jnp.float32)
        fsum_ref[0:1, :] = jnp.sum(out0_ref[...], axis=0, keepdims=True)
        fsum_ref[1:2, :] = jnp.zeros((1, fsum_ref.shape[1]), jnp.float32)

    @pl.when(phase == 1)
    def _layer1():
        acc = jnp.zeros(out1_ref.shape, jnp.float32)
        for r in range(nrel):
            acc += jnp.dot(adj_ref[r], g_ref[r], precision=_PREC,
                           preferred_element_type=jnp.float32)
        out1_ref[...] = acc
        fsum_ref[1:2, :] += jnp.sum(acc, axis=0, keepdims=True)


def kernel(adj_mat_list, bw0, bc0, bw1, bc1):
    nrel, n, _ = adj_mat_list.shape
    nb, _, h0 = bw0.shape
    h1 = bw1.shape[2]
    ni = n // _BM
    grid = (2, ni)

    out1, fsum = pl.pallas_call(
        _body,
        grid=grid,
        in_specs=[
            pl.BlockSpec(memory_space=pltpu.SMEM),
            pl.BlockSpec(memory_space=pltpu.SMEM),
            pl.BlockSpec((nrel, _BM, n), lambda p, i: (0, i, 0)),
            pl.BlockSpec((nb, n, h0), lambda p, i: (0, 0, 0)),
            pl.BlockSpec((nb, h0, h1), lambda p, i: (0, 0, 0)),
        ],
        out_specs=[
            pl.BlockSpec((_BM, h1), lambda p, i: (i, 0)),
            pl.BlockSpec((2, h0), lambda p, i: (0, 0)),
        ],
        out_shape=[
            jax.ShapeDtypeStruct((n, h1), jnp.float32),
            jax.ShapeDtypeStruct((2, h0), jnp.float32),
        ],
        scratch_shapes=[
            pltpu.VMEM((n, h0), jnp.float32),
            pltpu.VMEM((nrel, n, h0), jnp.float32),
        ],
        compiler_params=pltpu.CompilerParams(
            vmem_limit_bytes=63 * 1024 * 1024,
        ),
    )(bc0, bc1, adj_mat_list, bw0, bw1)

    final_rep = fsum.reshape(1, h0 + h1)
    return (out1, final_rep)
